# baseline (speedup 1.0000x reference)
import jax
import jax.numpy as jnp
from jax.experimental import pallas as pl


def _concat_body(xyz_ref, pts_ref, out_ref):
    out_ref[:, :, 0:3] = xyz_ref[...]
    out_ref[:, :, 3:] = pts_ref[...]


def kernel(xyz, points):
    B, N, C = xyz.shape
    D = points.shape[-1]
    NB = 4
    CHUNK = N // NB
    out = pl.pallas_call(
        _concat_body,
        grid=(B, NB),
        in_specs=[
            pl.BlockSpec((1, CHUNK, C), lambda b, n: (b, n, 0)),
            pl.BlockSpec((1, CHUNK, D), lambda b, n: (b, n, 0)),
        ],
        out_specs=pl.BlockSpec((1, CHUNK, C + D), lambda b, n: (b, n, 0)),
        out_shape=jax.ShapeDtypeStruct((B, N, C + D), xyz.dtype),
    )(xyz, points)
    new_xyz = jnp.zeros((B, 1, C), dtype=xyz.dtype)
    return new_xyz, out.reshape(B, 1, N, C + D)
